# Initial kernel scaffold; baseline (speedup 1.0000x reference)
import jax
import jax.numpy as jnp
from jax.experimental import pallas as pl


def kernel(inputs, embedding):
    raise NotImplementedError("write your pallas kernel here")


# speedup vs baseline: 5.5733x; 5.5733x over previous
import functools

import jax
import jax.numpy as jnp
from jax import lax
from jax.experimental import pallas as pl
from jax.experimental.pallas import tpu as pltpu
from jax.experimental.pallas import tpu_sc as plsc

V = 1000
D = 128
B = 1024
P = 64
C = 3
L = 16

NC = 2
NS = 16
NW = NC * NS

ND = 8
DC = D // ND
NB = NW // ND
BC = B // NB
G = 8
NG = BC // G

_mesh = plsc.VectorSubcoreMesh(core_axis_name="c", subcore_axis_name="s")


@functools.partial(
    pl.kernel,
    out_type=jax.ShapeDtypeStruct((B, D, P), jnp.float32),
    mesh=_mesh,
    compiler_params=pltpu.CompilerParams(
        use_tc_tiling_on_sc=False, needs_layout_passes=False),
    scratch_types=[
        pltpu.VMEM((DC // 2, C * V), jnp.int32),
        pltpu.VMEM((2, G, C, P), jnp.int32),
        pltpu.VMEM((2, G, DC, P), jnp.float32),
        pltpu.SemaphoreType.DMA((2,)),
        pltpu.SemaphoreType.DMA((2,)),
    ],
)
def _bow_kernel(tab_hbm, idx_hbm, out_hbm, tab_v, idx_v, out_v,
                idx_sem, out_sem):
    wid = lax.axis_index("s") * NC + lax.axis_index("c")
    d0 = (wid % ND) * DC
    b0 = (wid // ND) * BC

    pltpu.sync_copy(tab_hbm.at[pl.ds(d0 // 2, DC // 2)], tab_v)

    def idx_copy(g, k):
        return pltpu.make_async_copy(
            idx_hbm.at[pl.ds(b0 + g * G, G)], idx_v.at[k], idx_sem.at[k])

    def out_copy(g, k):
        return pltpu.make_async_copy(
            out_v.at[k],
            out_hbm.at[pl.ds(b0 + g * G, G), pl.ds(d0, DC)],
            out_sem.at[k])

    idx_copy(0, 0).start()
    idx_copy(1, 1).start()

    def outer(g0, carry):
        for k in range(2):
            g = g0 + k
            idx_copy(g, k).wait()
            @pl.when(g0 >= 2)
            def _():
                out_copy(g - 2, k).wait()
            def img(i, c2):
                for pg in range(P // L):
                    idx_c = []
                    for c in range(C):
                        iv = idx_v[k, i, c, pl.ds(pg * L, L)]
                        if c:
                            iv = iv + c * V
                        idx_c.append(iv)
                    for j in range(DC // 2):
                        row = jnp.full((16,), j, jnp.int32)
                        s = plsc.bitcast(
                            plsc.load_gather(tab_v, [row, idx_c[0]]),
                            jnp.bfloat16)
                        s = s + plsc.bitcast(
                            plsc.load_gather(tab_v, [row, idx_c[1]]),
                            jnp.bfloat16)
                        s = s + plsc.bitcast(
                            plsc.load_gather(tab_v, [row, idx_c[2]]),
                            jnp.bfloat16)
                        lo, hi = plsc.unpack(
                            s, format=plsc.PackFormat.INTERLEAVED)
                        out_v[k, i, 2 * j, pl.ds(pg * L, L)] = lo
                        out_v[k, i, 2 * j + 1, pl.ds(pg * L, L)] = hi
                return c2

            lax.fori_loop(0, G, img, None)
            out_copy(g, k).start()
            @pl.when(g + 2 < NG)
            def _():
                idx_copy(g + 2, k).start()
        return carry

    lax.fori_loop(0, NG // 2, lambda j, c: outer(j * 2, c), None,
                  unroll=False)

    out_copy(NG - 2, 0).wait()
    out_copy(NG - 1, 1).wait()


def kernel(inputs, embedding):
    b, h, w, c = inputs.shape
    tab_t = embedding.T.reshape(D, C * V)
    u = lax.bitcast_convert_type(
        tab_t.astype(jnp.bfloat16), jnp.uint16).astype(jnp.uint32)
    packed = lax.bitcast_convert_type(
        (u[1::2] << 16) | u[0::2], jnp.int32)
    idx = jnp.swapaxes(
        inputs.reshape(b, h * w, c).astype(jnp.int32), 1, 2)
    out = _bow_kernel(packed, idx)
    return out.reshape(b, D, h, w)
